# Initial kernel scaffold; baseline (speedup 1.0000x reference)
import jax
import jax.numpy as jnp
from jax.experimental import pallas as pl


def kernel(x, edge_index, W_l, b_l, W_r, gamma, beta):
    raise NotImplementedError("write your pallas kernel here")


# speedup vs baseline: 4.8317x; 4.8317x over previous
import functools
import math

import jax
import jax.numpy as jnp
from jax import lax
from jax.experimental import pallas as pl
from jax.experimental.pallas import tpu as pltpu
from jax.experimental.pallas import tpu_sc as plsc

N = 10000
E = 320000
D = 128

NC = 2
NS = 16
NW = NC * NS

CHUNK = 80
EDGES_PER_TILE = E // NW
NCHUNKS = EDGES_PER_TILE // CHUNK
STRIPE = 624
TAIL = N - NS * STRIPE
CW = 16
CPROWS = N * CW // 128
CPSTRIPE = 80
CPLAST = CPROWS - (NS - 1) * CPSTRIPE
CPPAD = 56
CPROWS_PAD = (NS - 1) * CPSTRIPE + CPPAD
CSTRIPE = 640
CBLK = 80
CBLK_FULL = CSTRIPE // CBLK
CBLK_LAST = (N - (NS - 1) * CSTRIPE) // CBLK

_mesh = plsc.VectorSubcoreMesh(
    core_axis_name="c", subcore_axis_name="s", num_cores=NC, num_subcores=NS
)


@functools.partial(
    pl.kernel,
    out_type=jax.ShapeDtypeStruct((NC * N, D), jnp.float32),
    mesh=_mesh,
    scratch_types=[
        pltpu.VMEM((CHUNK,), jnp.int32),
        pltpu.VMEM((CHUNK, D), jnp.float32),
        pltpu.VMEM_SHARED((N, D), jnp.float32),
        pltpu.SemaphoreType.DMA,
    ],
)
def _sc_aggregate(x_hbm, src_hbm, dst_hbm, agg_out, src_v, rows_v, agg_sh, sem):
    cid = lax.axis_index("c")
    sid = lax.axis_index("s")
    wid = cid * NS + sid

    zeros16 = jnp.zeros((16,), jnp.float32)

    @pl.loop(0, CHUNK)
    def _(i):
        @pl.loop(0, D // 16)
        def _(j):
            rows_v[i, pl.ds(j * 16, 16)] = zeros16

    row0 = sid * STRIPE
    nfull = STRIPE // CHUNK
    rem = STRIPE - nfull * CHUNK

    @pl.loop(0, nfull)
    def _(k):
        pltpu.sync_copy(rows_v, agg_sh.at[pl.ds(row0 + k * CHUNK, CHUNK)])

    pltpu.sync_copy(rows_v.at[pl.ds(0, rem)],
                    agg_sh.at[pl.ds(row0 + nfull * CHUNK, rem)])

    @pl.when(sid == NS - 1)
    def _():
        pltpu.sync_copy(rows_v.at[pl.ds(0, TAIL)],
                        agg_sh.at[pl.ds(NS * STRIPE, TAIL)])

    plsc.subcore_barrier()

    ebase = wid * EDGES_PER_TILE

    @pl.loop(0, NCHUNKS)
    def _(k):
        base = ebase + k * CHUNK
        pltpu.sync_copy(src_hbm.at[pl.ds(base, CHUNK)], src_v)
        pltpu.async_copy(x_hbm.at[src_v], rows_v, sem).wait()
        pltpu.sync_copy(src_hbm.at[pl.ds(E + base, CHUNK)], src_v)
        pltpu.sync_copy(rows_v, agg_sh.at[src_v], add=True)

    plsc.subcore_barrier()

    obase = cid * N + sid * STRIPE
    pltpu.sync_copy(agg_sh.at[pl.ds(sid * STRIPE, STRIPE)],
                    agg_out.at[pl.ds(obase, STRIPE)])

    @pl.when(sid == NS - 1)
    def _():
        pltpu.sync_copy(agg_sh.at[pl.ds(NS * STRIPE, TAIL)],
                        agg_out.at[pl.ds(cid * N + NS * STRIPE, TAIL)])


@functools.partial(
    pl.kernel,
    out_type=jax.ShapeDtypeStruct((NC * N, D), jnp.float32),
    mesh=_mesh,
    scratch_types=[
        pltpu.VMEM((CHUNK,), jnp.int32),
        pltpu.VMEM((CHUNK, D), jnp.float32),
        pltpu.VMEM_SHARED((N, D), jnp.float32),
        pltpu.SemaphoreType.DMA,
    ],
)
def _sc_count(dst_hbm, cnt_out, dst_v, ones_v, cnt_sh, sem):
    cid = lax.axis_index("c")
    sid = lax.axis_index("s")
    wid = cid * NS + sid

    zeros16 = jnp.zeros((16,), jnp.float32)
    ones16 = jnp.ones((16,), jnp.float32)

    @pl.loop(0, CHUNK)
    def _(i):
        @pl.loop(0, D // 16)
        def _(j):
            ones_v[i, pl.ds(j * 16, 16)] = zeros16

    row0 = sid * STRIPE
    nfull = STRIPE // CHUNK
    rem = STRIPE - nfull * CHUNK

    @pl.loop(0, nfull)
    def _(k):
        pltpu.sync_copy(ones_v, cnt_sh.at[pl.ds(row0 + k * CHUNK, CHUNK)])

    pltpu.sync_copy(ones_v.at[pl.ds(0, rem)],
                    cnt_sh.at[pl.ds(row0 + nfull * CHUNK, rem)])

    @pl.when(sid == NS - 1)
    def _():
        pltpu.sync_copy(ones_v.at[pl.ds(0, TAIL)],
                        cnt_sh.at[pl.ds(NS * STRIPE, TAIL)])

    @pl.loop(0, CHUNK)
    def _(i):
        ones_v[i, pl.ds(0, 16)] = ones16

    plsc.subcore_barrier()

    ebase = wid * EDGES_PER_TILE

    @pl.loop(0, NCHUNKS)
    def _(k):
        base = ebase + k * CHUNK
        pltpu.sync_copy(dst_hbm.at[pl.ds(base, CHUNK)], dst_v)
        pltpu.sync_copy(ones_v, cnt_sh.at[dst_v], add=True)

    plsc.subcore_barrier()

    obase = cid * N + sid * STRIPE
    pltpu.sync_copy(cnt_sh.at[pl.ds(sid * STRIPE, STRIPE)],
                    cnt_out.at[pl.ds(obase, STRIPE)])

    @pl.when(sid == NS - 1)
    def _():
        pltpu.sync_copy(cnt_sh.at[pl.ds(NS * STRIPE, TAIL)],
                        cnt_out.at[pl.ds(cid * N + NS * STRIPE, TAIL)])

BN = 2000
_INV_SQRT2 = 1.0 / math.sqrt(2.0)


def _tc_body(x_ref, a0_ref, a1_ref, c0_ref, c1_ref, wl_ref, wr_ref,
             bl_ref, g_ref, b_ref, o_ref):
    x = x_ref[...]
    agg = a0_ref[...] + a1_ref[...]
    cnt = c0_ref[...][:, :1] + c1_ref[...][:, :1]
    mean = agg / jnp.maximum(cnt, 1.0)
    f = (
        jnp.dot(mean, wl_ref[...], preferred_element_type=jnp.float32)
        + jnp.dot(x, wr_ref[...], preferred_element_type=jnp.float32)
        + bl_ref[...]
    )
    f = 0.5 * f * (1.0 + lax.erf(f * _INV_SQRT2))
    mu = jnp.mean(f, axis=1, keepdims=True)
    d = f - mu
    var = jnp.mean(d * d, axis=1, keepdims=True)
    o_ref[...] = d * lax.rsqrt(var + 1e-5) * g_ref[...] + b_ref[...] + x


def _tc_epilogue(x, agg_part, cnt_part, wl_t, wr_t, b_l, gamma, beta):
    nb = N // BN
    return pl.pallas_call(
        _tc_body,
        grid=(nb,),
        in_specs=[
            pl.BlockSpec((BN, D), lambda i: (i, 0)),
            pl.BlockSpec((BN, D), lambda i: (i, 0)),
            pl.BlockSpec((BN, D), lambda i: (i + nb, 0)),
            pl.BlockSpec((BN, D), lambda i: (i, 0)),
            pl.BlockSpec((BN, D), lambda i: (i + nb, 0)),
            pl.BlockSpec((D, D), lambda i: (0, 0)),
            pl.BlockSpec((D, D), lambda i: (0, 0)),
            pl.BlockSpec((1, D), lambda i: (0, 0)),
            pl.BlockSpec((1, D), lambda i: (0, 0)),
            pl.BlockSpec((1, D), lambda i: (0, 0)),
        ],
        out_specs=pl.BlockSpec((BN, D), lambda i: (i, 0)),
        out_shape=jax.ShapeDtypeStruct((N, D), jnp.float32),
    )(x, agg_part, agg_part, cnt_part, cnt_part, wl_t, wr_t, b_l, gamma, beta)


@jax.jit
def kernel(x, edge_index, W_l, b_l, W_r, gamma, beta):
    src = edge_index[0]
    dst = edge_index[1]
    se = edge_index.reshape(2 * E)
    agg_part = _sc_aggregate(x, se, dst)
    cnt_part = _sc_count(dst)
    return _tc_epilogue(
        x, agg_part, cnt_part,
        W_l.T, W_r.T,
        b_l.reshape(1, D), gamma.reshape(1, D), beta.reshape(1, D),
    )


# speedup vs baseline: 7.6996x; 1.5936x over previous
import functools
import math

import jax
import jax.numpy as jnp
from jax import lax
from jax.experimental import pallas as pl
from jax.experimental.pallas import tpu as pltpu
from jax.experimental.pallas import tpu_sc as plsc

N = 10000
E = 320000
D = 128

NC = 2
NS = 16
NW = NC * NS

CHUNK = 80
EDGES_PER_TILE = E // NW
NCHUNKS = EDGES_PER_TILE // CHUNK
STRIPE = 624
TAIL = N - NS * STRIPE
CW = 16
CPROWS = N * CW // 128
CPSTRIPE = 80
CPLAST = CPROWS - (NS - 1) * CPSTRIPE
CPPAD = 56
CPROWS_PAD = (NS - 1) * CPSTRIPE + CPPAD
CSTRIPE = 640
CBLK = 80
CBLK_FULL = CSTRIPE // CBLK
CBLK_LAST = (N - (NS - 1) * CSTRIPE) // CBLK

_mesh = plsc.VectorSubcoreMesh(
    core_axis_name="c", subcore_axis_name="s", num_cores=NC, num_subcores=NS
)


@functools.partial(
    pl.kernel,
    out_type=jax.ShapeDtypeStruct((NC * N, D), jnp.float32),
    mesh=_mesh,
    scratch_types=[
        pltpu.VMEM((CHUNK,), jnp.int32),
        pltpu.VMEM((CHUNK,), jnp.int32),
        pltpu.VMEM((CHUNK,), jnp.int32),
        pltpu.VMEM((CHUNK,), jnp.int32),
        pltpu.VMEM((CHUNK, D), jnp.float32),
        pltpu.VMEM((CHUNK, D), jnp.float32),
        pltpu.VMEM_SHARED((N, D), jnp.float32),
        pltpu.SemaphoreType.DMA,
        pltpu.SemaphoreType.DMA,
    ],
)
def _sc_aggregate(x_hbm, src_hbm, dst_hbm, agg_out,
                  srcA, dstA, srcB, dstB, rowsA, rowsB, agg_sh, semA, semB):
    cid = lax.axis_index("c")
    sid = lax.axis_index("s")
    wid = cid * NS + sid

    zeros16 = jnp.zeros((16,), jnp.float32)

    @pl.loop(0, CHUNK)
    def _(i):
        @pl.loop(0, D // 16)
        def _(j):
            rowsA[i, pl.ds(j * 16, 16)] = zeros16

    row0 = sid * STRIPE
    nfull = STRIPE // CHUNK
    rem = STRIPE - nfull * CHUNK

    @pl.loop(0, nfull)
    def _(k):
        pltpu.sync_copy(rowsA, agg_sh.at[pl.ds(row0 + k * CHUNK, CHUNK)])

    pltpu.sync_copy(rowsA.at[pl.ds(0, rem)],
                    agg_sh.at[pl.ds(row0 + nfull * CHUNK, rem)])

    @pl.when(sid == NS - 1)
    def _():
        pltpu.sync_copy(rowsA.at[pl.ds(0, TAIL)],
                        agg_sh.at[pl.ds(NS * STRIPE, TAIL)])

    plsc.subcore_barrier()

    ebase = wid * EDGES_PER_TILE

    def load_idx(c, sref, dref):
        base = ebase + c * CHUNK
        pltpu.sync_copy(src_hbm.at[pl.ds(base, CHUNK)], sref)
        pltpu.sync_copy(src_hbm.at[pl.ds(E + base, CHUNK)], dref)

    load_idx(0, srcA, dstA)
    pltpu.async_copy(x_hbm.at[srcA], rowsA, semA)

    @pl.loop(0, (NCHUNKS - 1) // 2)
    def _(i):
        a = 2 * i
        load_idx(a + 1, srcB, dstB)
        pltpu.async_copy(x_hbm.at[srcB], rowsB, semB)
        pltpu.make_async_copy(x_hbm.at[srcA], rowsA, semA).wait()
        pltpu.sync_copy(rowsA, agg_sh.at[dstA], add=True)
        load_idx(a + 2, srcA, dstA)
        pltpu.async_copy(x_hbm.at[srcA], rowsA, semA)
        pltpu.make_async_copy(x_hbm.at[srcB], rowsB, semB).wait()
        pltpu.sync_copy(rowsB, agg_sh.at[dstB], add=True)

    pltpu.make_async_copy(x_hbm.at[srcA], rowsA, semA).wait()
    pltpu.sync_copy(rowsA, agg_sh.at[dstA], add=True)

    plsc.subcore_barrier()

    obase = cid * N + sid * STRIPE
    pltpu.sync_copy(agg_sh.at[pl.ds(sid * STRIPE, STRIPE)],
                    agg_out.at[pl.ds(obase, STRIPE)])

    @pl.when(sid == NS - 1)
    def _():
        pltpu.sync_copy(agg_sh.at[pl.ds(NS * STRIPE, TAIL)],
                        agg_out.at[pl.ds(cid * N + NS * STRIPE, TAIL)])


@functools.partial(
    pl.kernel,
    out_type=jax.ShapeDtypeStruct((NC * N, D), jnp.float32),
    mesh=_mesh,
    scratch_types=[
        pltpu.VMEM((CHUNK,), jnp.int32),
        pltpu.VMEM((CHUNK,), jnp.int32),
        pltpu.VMEM((CHUNK, D), jnp.float32),
        pltpu.VMEM_SHARED((N, D), jnp.float32),
        pltpu.SemaphoreType.DMA,
        pltpu.SemaphoreType.DMA,
    ],
)
def _sc_count(dst_hbm, cnt_out, dstA, dstB, ones_v, cnt_sh, semA, semB):
    cid = lax.axis_index("c")
    sid = lax.axis_index("s")
    wid = cid * NS + sid

    zeros16 = jnp.zeros((16,), jnp.float32)
    ones16 = jnp.ones((16,), jnp.float32)

    @pl.loop(0, CHUNK)
    def _(i):
        @pl.loop(0, D // 16)
        def _(j):
            ones_v[i, pl.ds(j * 16, 16)] = zeros16

    row0 = sid * STRIPE
    nfull = STRIPE // CHUNK
    rem = STRIPE - nfull * CHUNK

    @pl.loop(0, nfull)
    def _(k):
        pltpu.sync_copy(ones_v, cnt_sh.at[pl.ds(row0 + k * CHUNK, CHUNK)])

    pltpu.sync_copy(ones_v.at[pl.ds(0, rem)],
                    cnt_sh.at[pl.ds(row0 + nfull * CHUNK, rem)])

    @pl.when(sid == NS - 1)
    def _():
        pltpu.sync_copy(ones_v.at[pl.ds(0, TAIL)],
                        cnt_sh.at[pl.ds(NS * STRIPE, TAIL)])

    @pl.loop(0, CHUNK)
    def _(i):
        ones_v[i, pl.ds(0, 16)] = ones16

    plsc.subcore_barrier()

    ebase = wid * EDGES_PER_TILE

    pltpu.sync_copy(dst_hbm.at[pl.ds(ebase, CHUNK)], dstA)

    @pl.loop(0, (NCHUNKS - 1) // 2)
    def _(i):
        a = 2 * i
        pltpu.async_copy(dst_hbm.at[pl.ds(ebase + (a + 1) * CHUNK, CHUNK)],
                         dstB, semB)
        pltpu.sync_copy(ones_v, cnt_sh.at[dstA], add=True)
        pltpu.make_async_copy(dst_hbm.at[pl.ds(ebase, CHUNK)], dstB, semB).wait()
        pltpu.async_copy(dst_hbm.at[pl.ds(ebase + (a + 2) * CHUNK, CHUNK)],
                         dstA, semA)
        pltpu.sync_copy(ones_v, cnt_sh.at[dstB], add=True)
        pltpu.make_async_copy(dst_hbm.at[pl.ds(ebase, CHUNK)], dstA, semA).wait()

    pltpu.sync_copy(ones_v, cnt_sh.at[dstA], add=True)

    plsc.subcore_barrier()

    obase = cid * N + sid * STRIPE
    pltpu.sync_copy(cnt_sh.at[pl.ds(sid * STRIPE, STRIPE)],
                    cnt_out.at[pl.ds(obase, STRIPE)])

    @pl.when(sid == NS - 1)
    def _():
        pltpu.sync_copy(cnt_sh.at[pl.ds(NS * STRIPE, TAIL)],
                        cnt_out.at[pl.ds(cid * N + NS * STRIPE, TAIL)])

BN = 2000
_INV_SQRT2 = 1.0 / math.sqrt(2.0)


def _tc_body(x_ref, a0_ref, a1_ref, c0_ref, c1_ref, wl_ref, wr_ref,
             bl_ref, g_ref, b_ref, o_ref):
    x = x_ref[...]
    agg = a0_ref[...] + a1_ref[...]
    cnt = c0_ref[...][:, :1] + c1_ref[...][:, :1]
    mean = agg / jnp.maximum(cnt, 1.0)
    f = (
        jnp.dot(mean, wl_ref[...], preferred_element_type=jnp.float32)
        + jnp.dot(x, wr_ref[...], preferred_element_type=jnp.float32)
        + bl_ref[...]
    )
    f = 0.5 * f * (1.0 + lax.erf(f * _INV_SQRT2))
    mu = jnp.mean(f, axis=1, keepdims=True)
    d = f - mu
    var = jnp.mean(d * d, axis=1, keepdims=True)
    o_ref[...] = d * lax.rsqrt(var + 1e-5) * g_ref[...] + b_ref[...] + x


def _tc_epilogue(x, agg_part, cnt_part, wl_t, wr_t, b_l, gamma, beta):
    nb = N // BN
    return pl.pallas_call(
        _tc_body,
        grid=(nb,),
        in_specs=[
            pl.BlockSpec((BN, D), lambda i: (i, 0)),
            pl.BlockSpec((BN, D), lambda i: (i, 0)),
            pl.BlockSpec((BN, D), lambda i: (i + nb, 0)),
            pl.BlockSpec((BN, D), lambda i: (i, 0)),
            pl.BlockSpec((BN, D), lambda i: (i + nb, 0)),
            pl.BlockSpec((D, D), lambda i: (0, 0)),
            pl.BlockSpec((D, D), lambda i: (0, 0)),
            pl.BlockSpec((1, D), lambda i: (0, 0)),
            pl.BlockSpec((1, D), lambda i: (0, 0)),
            pl.BlockSpec((1, D), lambda i: (0, 0)),
        ],
        out_specs=pl.BlockSpec((BN, D), lambda i: (i, 0)),
        out_shape=jax.ShapeDtypeStruct((N, D), jnp.float32),
    )(x, agg_part, agg_part, cnt_part, cnt_part, wl_t, wr_t, b_l, gamma, beta)


@jax.jit
def kernel(x, edge_index, W_l, b_l, W_r, gamma, beta):
    src = edge_index[0]
    dst = edge_index[1]
    se = edge_index.reshape(2 * E)
    agg_part = _sc_aggregate(x, se, dst)
    cnt_part = _sc_count(dst)
    return _tc_epilogue(
        x, agg_part, cnt_part,
        W_l.T, W_r.T,
        b_l.reshape(1, D), gamma.reshape(1, D), beta.reshape(1, D),
    )


# speedup vs baseline: 7.7152x; 1.0020x over previous
import functools
import math

import jax
import jax.numpy as jnp
from jax import lax
from jax.experimental import pallas as pl
from jax.experimental.pallas import tpu as pltpu
from jax.experimental.pallas import tpu_sc as plsc

N = 10000
E = 320000
D = 128

NC = 2
NS = 16
NW = NC * NS

CHUNK = 80
EDGES_PER_TILE = E // NW
NCHUNKS = EDGES_PER_TILE // CHUNK
STRIPE = 624
TAIL = N - NS * STRIPE
NBUF = 4

_mesh = plsc.VectorSubcoreMesh(
    core_axis_name="c", subcore_axis_name="s", num_cores=NC, num_subcores=NS
)


@functools.partial(
    pl.kernel,
    out_type=jax.ShapeDtypeStruct((NC * N, D), jnp.float32),
    mesh=_mesh,
    scratch_types=(
        [pltpu.VMEM((CHUNK,), jnp.int32) for _ in range(2 * NBUF)]
        + [pltpu.VMEM((CHUNK, D), jnp.float32) for _ in range(NBUF)]
        + [pltpu.VMEM_SHARED((N, D), jnp.float32)]
        + [pltpu.SemaphoreType.DMA for _ in range(NBUF)]
    ),
)
def _sc_aggregate(x_hbm, src_hbm, dst_hbm, agg_out, *scratch):
    srcs = scratch[0:NBUF]
    dsts = scratch[NBUF:2 * NBUF]
    rows = scratch[2 * NBUF:3 * NBUF]
    agg_sh = scratch[3 * NBUF]
    sems = scratch[3 * NBUF + 1:]
    rowsA = rows[0]
    cid = lax.axis_index("c")
    sid = lax.axis_index("s")
    wid = cid * NS + sid

    zeros16 = jnp.zeros((16,), jnp.float32)

    @pl.loop(0, CHUNK)
    def _(i):
        @pl.loop(0, D // 16)
        def _(j):
            rowsA[i, pl.ds(j * 16, 16)] = zeros16

    row0 = sid * STRIPE
    nfull = STRIPE // CHUNK
    rem = STRIPE - nfull * CHUNK

    @pl.loop(0, nfull)
    def _(k):
        pltpu.sync_copy(rowsA, agg_sh.at[pl.ds(row0 + k * CHUNK, CHUNK)])

    pltpu.sync_copy(rowsA.at[pl.ds(0, rem)],
                    agg_sh.at[pl.ds(row0 + nfull * CHUNK, rem)])

    @pl.when(sid == NS - 1)
    def _():
        pltpu.sync_copy(rowsA.at[pl.ds(0, TAIL)],
                        agg_sh.at[pl.ds(NS * STRIPE, TAIL)])

    plsc.subcore_barrier()

    ebase = wid * EDGES_PER_TILE

    def load_idx(c, b):
        base = ebase + c * CHUNK
        pltpu.sync_copy(src_hbm.at[pl.ds(base, CHUNK)], srcs[b])
        pltpu.sync_copy(src_hbm.at[pl.ds(E + base, CHUNK)], dsts[b])

    def fire(c, b):
        load_idx(c, b)
        pltpu.async_copy(x_hbm.at[srcs[b]], rows[b], sems[b])

    def drain(b):
        pltpu.make_async_copy(x_hbm.at[srcs[b]], rows[b], sems[b]).wait()
        pltpu.sync_copy(rows[b], agg_sh.at[dsts[b]], add=True)

    for b in range(NBUF - 1):
        fire(b, b)

    @pl.loop(0, NCHUNKS // NBUF)
    def _(g):
        c0 = g * NBUF
        for b in range(NBUF):
            c = c0 + b

            @pl.when(c + NBUF - 1 < NCHUNKS)
            def _():
                fire(c + NBUF - 1, (b + NBUF - 1) % NBUF)

            drain(b)

    for b in range(NCHUNKS % NBUF):
        bb = (NCHUNKS // NBUF * NBUF + b) % NBUF
        drain(bb)

    plsc.subcore_barrier()

    obase = cid * N + sid * STRIPE
    pltpu.sync_copy(agg_sh.at[pl.ds(sid * STRIPE, STRIPE)],
                    agg_out.at[pl.ds(obase, STRIPE)])

    @pl.when(sid == NS - 1)
    def _():
        pltpu.sync_copy(agg_sh.at[pl.ds(NS * STRIPE, TAIL)],
                        agg_out.at[pl.ds(cid * N + NS * STRIPE, TAIL)])


@functools.partial(
    pl.kernel,
    out_type=jax.ShapeDtypeStruct((NC * N, D), jnp.float32),
    mesh=_mesh,
    scratch_types=[
        pltpu.VMEM((CHUNK,), jnp.int32),
        pltpu.VMEM((CHUNK,), jnp.int32),
        pltpu.VMEM((CHUNK, D), jnp.float32),
        pltpu.VMEM_SHARED((N, D), jnp.float32),
        pltpu.SemaphoreType.DMA,
        pltpu.SemaphoreType.DMA,
    ],
)
def _sc_count(dst_hbm, cnt_out, dstA, dstB, ones_v, cnt_sh, semA, semB):
    cid = lax.axis_index("c")
    sid = lax.axis_index("s")
    wid = cid * NS + sid

    zeros16 = jnp.zeros((16,), jnp.float32)
    ones16 = jnp.ones((16,), jnp.float32)

    @pl.loop(0, CHUNK)
    def _(i):
        @pl.loop(0, D // 16)
        def _(j):
            ones_v[i, pl.ds(j * 16, 16)] = zeros16

    row0 = sid * STRIPE
    nfull = STRIPE // CHUNK
    rem = STRIPE - nfull * CHUNK

    @pl.loop(0, nfull)
    def _(k):
        pltpu.sync_copy(ones_v, cnt_sh.at[pl.ds(row0 + k * CHUNK, CHUNK)])

    pltpu.sync_copy(ones_v.at[pl.ds(0, rem)],
                    cnt_sh.at[pl.ds(row0 + nfull * CHUNK, rem)])

    @pl.when(sid == NS - 1)
    def _():
        pltpu.sync_copy(ones_v.at[pl.ds(0, TAIL)],
                        cnt_sh.at[pl.ds(NS * STRIPE, TAIL)])

    @pl.loop(0, CHUNK)
    def _(i):
        ones_v[i, pl.ds(0, 16)] = ones16

    plsc.subcore_barrier()

    ebase = wid * EDGES_PER_TILE

    pltpu.sync_copy(dst_hbm.at[pl.ds(ebase, CHUNK)], dstA)

    @pl.loop(0, (NCHUNKS - 1) // 2)
    def _(i):
        a = 2 * i
        pltpu.async_copy(dst_hbm.at[pl.ds(ebase + (a + 1) * CHUNK, CHUNK)],
                         dstB, semB)
        pltpu.sync_copy(ones_v, cnt_sh.at[dstA], add=True)
        pltpu.make_async_copy(dst_hbm.at[pl.ds(ebase, CHUNK)], dstB, semB).wait()
        pltpu.async_copy(dst_hbm.at[pl.ds(ebase + (a + 2) * CHUNK, CHUNK)],
                         dstA, semA)
        pltpu.sync_copy(ones_v, cnt_sh.at[dstB], add=True)
        pltpu.make_async_copy(dst_hbm.at[pl.ds(ebase, CHUNK)], dstA, semA).wait()

    pltpu.sync_copy(ones_v, cnt_sh.at[dstA], add=True)

    plsc.subcore_barrier()

    obase = cid * N + sid * STRIPE
    pltpu.sync_copy(cnt_sh.at[pl.ds(sid * STRIPE, STRIPE)],
                    cnt_out.at[pl.ds(obase, STRIPE)])

    @pl.when(sid == NS - 1)
    def _():
        pltpu.sync_copy(cnt_sh.at[pl.ds(NS * STRIPE, TAIL)],
                        cnt_out.at[pl.ds(cid * N + NS * STRIPE, TAIL)])

BN = 2000
_INV_SQRT2 = 1.0 / math.sqrt(2.0)


def _tc_body(x_ref, a0_ref, a1_ref, c0_ref, c1_ref, wl_ref, wr_ref,
             bl_ref, g_ref, b_ref, o_ref):
    x = x_ref[...]
    agg = a0_ref[...] + a1_ref[...]
    cnt = c0_ref[...][:, :1] + c1_ref[...][:, :1]
    mean = agg / jnp.maximum(cnt, 1.0)
    f = (
        jnp.dot(mean, wl_ref[...], preferred_element_type=jnp.float32)
        + jnp.dot(x, wr_ref[...], preferred_element_type=jnp.float32)
        + bl_ref[...]
    )
    f = 0.5 * f * (1.0 + lax.erf(f * _INV_SQRT2))
    mu = jnp.mean(f, axis=1, keepdims=True)
    d = f - mu
    var = jnp.mean(d * d, axis=1, keepdims=True)
    o_ref[...] = d * lax.rsqrt(var + 1e-5) * g_ref[...] + b_ref[...] + x


def _tc_epilogue(x, agg_part, cnt_part, wl_t, wr_t, b_l, gamma, beta):
    nb = N // BN
    return pl.pallas_call(
        _tc_body,
        grid=(nb,),
        in_specs=[
            pl.BlockSpec((BN, D), lambda i: (i, 0)),
            pl.BlockSpec((BN, D), lambda i: (i, 0)),
            pl.BlockSpec((BN, D), lambda i: (i + nb, 0)),
            pl.BlockSpec((BN, D), lambda i: (i, 0)),
            pl.BlockSpec((BN, D), lambda i: (i + nb, 0)),
            pl.BlockSpec((D, D), lambda i: (0, 0)),
            pl.BlockSpec((D, D), lambda i: (0, 0)),
            pl.BlockSpec((1, D), lambda i: (0, 0)),
            pl.BlockSpec((1, D), lambda i: (0, 0)),
            pl.BlockSpec((1, D), lambda i: (0, 0)),
        ],
        out_specs=pl.BlockSpec((BN, D), lambda i: (i, 0)),
        out_shape=jax.ShapeDtypeStruct((N, D), jnp.float32),
    )(x, agg_part, agg_part, cnt_part, cnt_part, wl_t, wr_t, b_l, gamma, beta)


@jax.jit
def kernel(x, edge_index, W_l, b_l, W_r, gamma, beta):
    src = edge_index[0]
    dst = edge_index[1]
    se = edge_index.reshape(2 * E)
    agg_part = _sc_aggregate(x, se, dst)
    cnt_part = _sc_count(dst)
    return _tc_epilogue(
        x, agg_part, cnt_part,
        W_l.T, W_r.T,
        b_l.reshape(1, D), gamma.reshape(1, D), beta.reshape(1, D),
    )


# speedup vs baseline: 8.7215x; 1.1304x over previous
import functools
import math

import jax
import jax.numpy as jnp
from jax import lax
from jax.experimental import pallas as pl
from jax.experimental.pallas import tpu as pltpu
from jax.experimental.pallas import tpu_sc as plsc

N = 10000
E = 320000
D = 128

NC = 2
NS = 16
NW = NC * NS

CHUNK = 128
NCH_LO = E // NW // CHUNK
XTRA = E // CHUNK - NW * NCH_LO
NCH_HI = NCH_LO + 1
STRIPE = 624
TAIL = N - NS * STRIPE
NBUF = 2

_mesh = plsc.VectorSubcoreMesh(
    core_axis_name="c", subcore_axis_name="s", num_cores=NC, num_subcores=NS
)


@functools.partial(
    pl.kernel,
    out_type=jax.ShapeDtypeStruct((NC * N, D), jnp.float32),
    mesh=_mesh,
    scratch_types=(
        [pltpu.VMEM((CHUNK,), jnp.int32) for _ in range(2 * NBUF)]
        + [pltpu.VMEM((CHUNK, D), jnp.float32) for _ in range(NBUF)]
        + [pltpu.VMEM_SHARED((N, D), jnp.float32)]
        + [pltpu.SemaphoreType.DMA for _ in range(NBUF)]
    ),
)
def _sc_aggregate(x_hbm, src_hbm, dst_hbm, agg_out, *scratch):
    srcs = scratch[0:NBUF]
    dsts = scratch[NBUF:2 * NBUF]
    rows = scratch[2 * NBUF:3 * NBUF]
    agg_sh = scratch[3 * NBUF]
    sems = scratch[3 * NBUF + 1:]
    rowsA = rows[0]
    cid = lax.axis_index("c")
    sid = lax.axis_index("s")
    wid = cid * NS + sid

    zeros16 = jnp.zeros((16,), jnp.float32)

    @pl.loop(0, CHUNK)
    def _(i):
        @pl.loop(0, D // 16)
        def _(j):
            rowsA[i, pl.ds(j * 16, 16)] = zeros16

    row0 = sid * STRIPE
    nfull = STRIPE // CHUNK
    rem = STRIPE - nfull * CHUNK

    @pl.loop(0, nfull)
    def _(k):
        pltpu.sync_copy(rowsA, agg_sh.at[pl.ds(row0 + k * CHUNK, CHUNK)])

    pltpu.sync_copy(rowsA.at[pl.ds(0, rem)],
                    agg_sh.at[pl.ds(row0 + nfull * CHUNK, rem)])

    @pl.when(sid == NS - 1)
    def _():
        pltpu.sync_copy(rowsA.at[pl.ds(0, TAIL)],
                        agg_sh.at[pl.ds(NS * STRIPE, TAIL)])

    plsc.subcore_barrier()

    nch = lax.select(wid < XTRA, NCH_HI, NCH_LO)
    ebase = (wid * NCH_LO + lax.min(wid, XTRA)) * CHUNK

    def load_idx(c, b):
        base = ebase + c * CHUNK
        pltpu.sync_copy(src_hbm.at[pl.ds(base, CHUNK)], srcs[b])
        pltpu.sync_copy(src_hbm.at[pl.ds(E + base, CHUNK)], dsts[b])

    def fire(c, b):
        load_idx(c, b)
        pltpu.async_copy(x_hbm.at[srcs[b]], rows[b], sems[b])

    def drain(b):
        pltpu.make_async_copy(x_hbm.at[srcs[b]], rows[b], sems[b]).wait()
        pltpu.sync_copy(rows[b], agg_sh.at[dsts[b]], add=True)

    for b in range(NBUF - 1):
        fire(b, b)

    @pl.loop(0, (NCH_HI + NBUF - 1) // NBUF)
    def _(g):
        c0 = g * NBUF
        for b in range(NBUF):
            c = c0 + b

            @pl.when(c + NBUF - 1 < nch)
            def _():
                fire(c + NBUF - 1, (b + NBUF - 1) % NBUF)

            @pl.when(c < nch)
            def _():
                drain(b)

    plsc.subcore_barrier()

    obase = cid * N + sid * STRIPE
    pltpu.sync_copy(agg_sh.at[pl.ds(sid * STRIPE, STRIPE)],
                    agg_out.at[pl.ds(obase, STRIPE)])

    @pl.when(sid == NS - 1)
    def _():
        pltpu.sync_copy(agg_sh.at[pl.ds(NS * STRIPE, TAIL)],
                        agg_out.at[pl.ds(cid * N + NS * STRIPE, TAIL)])


@functools.partial(
    pl.kernel,
    out_type=jax.ShapeDtypeStruct((NC * N, D), jnp.float32),
    mesh=_mesh,
    scratch_types=[
        pltpu.VMEM((CHUNK,), jnp.int32),
        pltpu.VMEM((CHUNK,), jnp.int32),
        pltpu.VMEM((CHUNK, D), jnp.float32),
        pltpu.VMEM_SHARED((N, D), jnp.float32),
        pltpu.SemaphoreType.DMA,
        pltpu.SemaphoreType.DMA,
    ],
)
def _sc_count(dst_hbm, cnt_out, dstA, dstB, ones_v, cnt_sh, semA, semB):
    cid = lax.axis_index("c")
    sid = lax.axis_index("s")
    wid = cid * NS + sid

    zeros16 = jnp.zeros((16,), jnp.float32)
    ones16 = jnp.ones((16,), jnp.float32)

    @pl.loop(0, CHUNK)
    def _(i):
        @pl.loop(0, D // 16)
        def _(j):
            ones_v[i, pl.ds(j * 16, 16)] = zeros16

    row0 = sid * STRIPE
    nfull = STRIPE // CHUNK
    rem = STRIPE - nfull * CHUNK

    @pl.loop(0, nfull)
    def _(k):
        pltpu.sync_copy(ones_v, cnt_sh.at[pl.ds(row0 + k * CHUNK, CHUNK)])

    pltpu.sync_copy(ones_v.at[pl.ds(0, rem)],
                    cnt_sh.at[pl.ds(row0 + nfull * CHUNK, rem)])

    @pl.when(sid == NS - 1)
    def _():
        pltpu.sync_copy(ones_v.at[pl.ds(0, TAIL)],
                        cnt_sh.at[pl.ds(NS * STRIPE, TAIL)])

    @pl.loop(0, CHUNK)
    def _(i):
        ones_v[i, pl.ds(0, 16)] = ones16

    plsc.subcore_barrier()

    nch = lax.select(wid < XTRA, NCH_HI, NCH_LO)
    ebase = (wid * NCH_LO + lax.min(wid, XTRA)) * CHUNK

    pltpu.sync_copy(dst_hbm.at[pl.ds(ebase, CHUNK)], dstA)

    @pl.loop(0, (NCH_HI + 1) // 2)
    def _(i):
        a = 2 * i

        @pl.when(a + 1 < nch)
        def _():
            pltpu.async_copy(dst_hbm.at[pl.ds(ebase + (a + 1) * CHUNK, CHUNK)],
                             dstB, semB)

        @pl.when(a < nch)
        def _():
            pltpu.sync_copy(ones_v, cnt_sh.at[dstA], add=True)

        @pl.when(a + 1 < nch)
        def _():
            pltpu.make_async_copy(dst_hbm.at[pl.ds(ebase, CHUNK)], dstB,
                                  semB).wait()

        @pl.when(a + 2 < nch)
        def _():
            pltpu.async_copy(dst_hbm.at[pl.ds(ebase + (a + 2) * CHUNK, CHUNK)],
                             dstA, semA)

        @pl.when(a + 1 < nch)
        def _():
            pltpu.sync_copy(ones_v, cnt_sh.at[dstB], add=True)

        @pl.when(a + 2 < nch)
        def _():
            pltpu.make_async_copy(dst_hbm.at[pl.ds(ebase, CHUNK)], dstA,
                                  semA).wait()

    plsc.subcore_barrier()

    obase = cid * N + sid * STRIPE
    pltpu.sync_copy(cnt_sh.at[pl.ds(sid * STRIPE, STRIPE)],
                    cnt_out.at[pl.ds(obase, STRIPE)])

    @pl.when(sid == NS - 1)
    def _():
        pltpu.sync_copy(cnt_sh.at[pl.ds(NS * STRIPE, TAIL)],
                        cnt_out.at[pl.ds(cid * N + NS * STRIPE, TAIL)])

BN = 2000
_INV_SQRT2 = 1.0 / math.sqrt(2.0)


def _tc_body(x_ref, a0_ref, a1_ref, c0_ref, c1_ref, wl_ref, wr_ref,
             bl_ref, g_ref, b_ref, o_ref):
    x = x_ref[...]
    agg = a0_ref[...] + a1_ref[...]
    cnt = c0_ref[...][:, :1] + c1_ref[...][:, :1]
    mean = agg / jnp.maximum(cnt, 1.0)
    dn = (((1,), (1,)), ((), ()))
    f = (
        lax.dot_general(mean, wl_ref[...], dn,
                        preferred_element_type=jnp.float32)
        + lax.dot_general(x, wr_ref[...], dn,
                          preferred_element_type=jnp.float32)
        + bl_ref[...]
    )
    f = 0.5 * f * (1.0 + lax.erf(f * _INV_SQRT2))
    mu = jnp.mean(f, axis=1, keepdims=True)
    d = f - mu
    var = jnp.mean(d * d, axis=1, keepdims=True)
    o_ref[...] = d * lax.rsqrt(var + 1e-5) * g_ref[...] + b_ref[...] + x


def _tc_epilogue(x, agg_part, cnt_part, wl_t, wr_t, b_l, gamma, beta):
    nb = N // BN
    return pl.pallas_call(
        _tc_body,
        grid=(nb,),
        in_specs=[
            pl.BlockSpec((BN, D), lambda i: (i, 0)),
            pl.BlockSpec((BN, D), lambda i: (i, 0)),
            pl.BlockSpec((BN, D), lambda i: (i + nb, 0)),
            pl.BlockSpec((BN, D), lambda i: (i, 0)),
            pl.BlockSpec((BN, D), lambda i: (i + nb, 0)),
            pl.BlockSpec((D, D), lambda i: (0, 0)),
            pl.BlockSpec((D, D), lambda i: (0, 0)),
            pl.BlockSpec((1, D), lambda i: (0, 0)),
            pl.BlockSpec((1, D), lambda i: (0, 0)),
            pl.BlockSpec((1, D), lambda i: (0, 0)),
        ],
        out_specs=pl.BlockSpec((BN, D), lambda i: (i, 0)),
        out_shape=jax.ShapeDtypeStruct((N, D), jnp.float32),
    )(x, agg_part, agg_part, cnt_part, cnt_part, wl_t, wr_t, b_l, gamma, beta)


@jax.jit
def kernel(x, edge_index, W_l, b_l, W_r, gamma, beta):
    src = edge_index[0]
    dst = edge_index[1]
    se = edge_index.reshape(2 * E)
    agg_part = _sc_aggregate(x, se, dst)
    cnt_part = _sc_count(dst)
    return _tc_epilogue(
        x, agg_part, cnt_part,
        W_l, W_r,
        b_l.reshape(1, D), gamma.reshape(1, D), beta.reshape(1, D),
    )
